# baseline (speedup 1.0000x reference)
import math

import jax
import jax.numpy as jnp
from jax.experimental import pallas as pl
from jax.experimental.pallas import tpu as pltpu

B, C = 128, 8192
NEG_SCALE = 5.0
INIT_BIAS = 0.1
MAP_ALPHA, MAP_BETA, MAP_GAMMA = 10.0, 0.2, 0.1
FOCAL_GAMMA = 2.0
BALANCE_PARAM = 2.0
LOSS_WEIGHT = 1.0

CLEAN_RATE = 0.9
K_TOTAL = math.ceil(B * C * (1.0 - CLEAN_RATE))
P_K_MAX = math.ceil(K_TOTAL * 0.1)


def _main_kernel(tn_ref, score_ref, label_ref, cf_ref,
                 out_final_ref, out_loss_ref,
                 loss_ref, corr_ref, key_ref):
    score = score_ref[...]
    lab_i = label_ref[...]
    cf = cf_ref[...]
    tn = tn_ref[0, 0]

    init_bias = -jnp.log(tn / cf - 1.0) * (INIT_BIAS / NEG_SCALE)
    freq_inv = 1.0 / cf
    labf = jnp.maximum(lab_i, 0).astype(jnp.float32)

    def loss_an(sb, lab):
        rr = jnp.sum(lab * freq_inv, axis=1, keepdims=True)
        pw = freq_inv / rr
        w = jax.nn.sigmoid(MAP_BETA * (pw - MAP_GAMMA)) + MAP_ALPHA
        logits = sb * (1.0 - lab) * NEG_SCALE + sb * lab
        w = w / NEG_SCALE * (1.0 - lab) + w * lab
        bce = (jnp.maximum(logits, 0.0) - logits * lab
               + jnp.log1p(jnp.exp(-jnp.abs(logits))))
        pt = jnp.exp(-bce)
        om = 1.0 - pt
        return (LOSS_WEIGHT * BALANCE_PARAM) * (om * om * (w * bce))

    s1 = score + init_bias
    loss = loss_an(s1, labf)
    corr = loss_an(s1 + init_bias, 1.0 - labf)
    loss_ref[...] = loss
    corr_ref[...] = corr
    bits = jax.lax.bitcast_convert_type(loss, jnp.int32)
    neg0 = lab_i == 0
    key_ref[...] = jnp.where(neg0, bits, -bits)

    pos_f = jnp.sum(labf)
    p_k_f = jnp.minimum(jnp.float32(P_K_MAX), pos_f)
    n_k_f = jnp.float32(K_TOTAL) - p_k_f

    def body(i, carry):
        sel_n, sel_p = carry
        hi = jax.lax.shift_left(jnp.int32(1), 30 - 2 * i)
        lo = jax.lax.shift_right_logical(hi, 1)
        key = key_ref[...]

        def step2(sel, k_f, sign):
            c1 = jax.lax.bitwise_or(sel, lo)
            c2 = jax.lax.bitwise_or(sel, hi)
            c3 = jax.lax.bitwise_or(c2, lo)
            if sign > 0:
                i1 = jnp.where(key >= c1, 1.0, 0.0)
                i2 = jnp.where(key >= c2, 1.0, 0.0)
                i3 = jnp.where(key >= c3, 1.0, 0.0)
            else:
                i1 = jnp.where(key <= -c1, 1.0, 0.0)
                i2 = jnp.where(key <= -c2, 1.0, 0.0)
                i3 = jnp.where(key <= -c3, 1.0, 0.0)
            sel = jnp.where(jnp.sum(i1) >= k_f, c1, sel)
            sel = jnp.where(jnp.sum(i2) >= k_f, c2, sel)
            sel = jnp.where(jnp.sum(i3) >= k_f, c3, sel)
            return sel

        return step2(sel_n, n_k_f, 1), step2(sel_p, p_k_f, -1)

    sel_n, sel_p = jax.lax.fori_loop(
        0, 15, body, (jnp.int32(0), jnp.int32(0)))
    key = key_ref[...]
    c_n = jax.lax.bitwise_or(sel_n, jnp.int32(1))
    c_p = jax.lax.bitwise_or(sel_p, jnp.int32(1))
    cnt_n = jnp.sum(jnp.where(key >= c_n, 1.0, 0.0))
    cnt_p = jnp.sum(jnp.where(key <= -c_p, 1.0, 0.0))
    sel_n = jnp.where(cnt_n >= n_k_f, c_n, sel_n)
    sel_p = jnp.where(cnt_p >= p_k_f, c_p, sel_p)
    thr_n = jax.lax.bitcast_convert_type(sel_n, jnp.float32)
    thr_p = jax.lax.bitcast_convert_type(sel_p, jnp.float32)

    loss2 = loss_ref[...]
    corr2 = corr_ref[...]
    neg = label_ref[...] == 0
    u0 = jnp.where(neg, loss2, 0.0)
    u1 = jnp.where(neg, 0.0, loss2)
    keep = (u0 < thr_n) & (u1 < thr_p)
    final = jnp.where(keep, loss2, corr2)
    out_final_ref[0, 0] = jnp.sum(final)
    out_loss_ref[0, 0] = jnp.sum(loss2)


def kernel(cls_score, label, class_freq, neg_class_freq, epoch=1):
    train_num = (class_freq[0] + neg_class_freq[0]).reshape(1, 1)
    cf = class_freq.reshape(1, C)
    sums = pl.pallas_call(
        _main_kernel,
        out_shape=[
            jax.ShapeDtypeStruct((1, 1), jnp.float32),
            jax.ShapeDtypeStruct((1, 1), jnp.float32),
        ],
        in_specs=[
            pl.BlockSpec(memory_space=pltpu.SMEM),
            pl.BlockSpec(memory_space=pltpu.VMEM),
            pl.BlockSpec(memory_space=pltpu.VMEM),
            pl.BlockSpec(memory_space=pltpu.VMEM),
        ],
        out_specs=[
            pl.BlockSpec(memory_space=pltpu.SMEM),
            pl.BlockSpec(memory_space=pltpu.SMEM),
        ],
        scratch_shapes=[
            pltpu.VMEM((B, C), jnp.float32),
            pltpu.VMEM((B, C), jnp.float32),
            pltpu.VMEM((B, C), jnp.int32),
        ],
        compiler_params=pltpu.CompilerParams(
            vmem_limit_bytes=100 * 1024 * 1024,
        ),
    )(train_num, cls_score, label, cf)
    inv_n = 1.0 / float(B * C)
    mean_final = sums[0][0, 0] * inv_n
    mean_loss = sums[1][0, 0] * inv_n
    return jnp.where(epoch == 0, mean_loss, mean_final)


# speedup vs baseline: 1.1040x; 1.1040x over previous
import math

import jax
import jax.numpy as jnp
from jax.experimental import pallas as pl
from jax.experimental.pallas import tpu as pltpu

B, C = 128, 8192
NEG_SCALE = 5.0
INIT_BIAS = 0.1
MAP_ALPHA, MAP_BETA, MAP_GAMMA = 10.0, 0.2, 0.1
FOCAL_GAMMA = 2.0
BALANCE_PARAM = 2.0
LOSS_WEIGHT = 1.0

CLEAN_RATE = 0.9
K_TOTAL = math.ceil(B * C * (1.0 - CLEAN_RATE))
P_K_MAX = math.ceil(K_TOTAL * 0.1)


def _main_kernel(tn_ref, score_ref, label_ref, cf_ref,
                 out_final_ref, out_loss_ref,
                 loss_ref, corr_ref, key_ref, s16n_ref, s16p_ref):
    score = score_ref[...]
    lab_i = label_ref[...]
    cf = cf_ref[...]
    tn = tn_ref[0, 0]

    init_bias = -jnp.log(tn / cf - 1.0) * (INIT_BIAS / NEG_SCALE)
    freq_inv = 1.0 / cf
    labf = jnp.maximum(lab_i, 0).astype(jnp.float32)

    def loss_an(sb, lab):
        rr = jnp.sum(lab * freq_inv, axis=1, keepdims=True)
        pw = freq_inv / rr
        w = jax.nn.sigmoid(MAP_BETA * (pw - MAP_GAMMA)) + MAP_ALPHA
        logits = sb * (1.0 - lab) * NEG_SCALE + sb * lab
        w = w / NEG_SCALE * (1.0 - lab) + w * lab
        bce = (jnp.maximum(logits, 0.0) - logits * lab
               + jnp.log1p(jnp.exp(-jnp.abs(logits))))
        pt = jnp.exp(-bce)
        om = 1.0 - pt
        return (LOSS_WEIGHT * BALANCE_PARAM) * (om * om * (w * bce))

    s1 = score + init_bias
    loss = loss_an(s1, labf)
    corr = loss_an(s1 + init_bias, 1.0 - labf)
    loss_ref[...] = loss
    corr_ref[...] = corr
    bits = jax.lax.bitcast_convert_type(loss, jnp.int32)
    neg0 = lab_i == 0
    key_ref[...] = jnp.where(neg0, bits, -bits)

    pos_f = jnp.sum(labf)
    p_k_f = jnp.minimum(jnp.float32(P_K_MAX), pos_f)
    n_k_f = jnp.float32(K_TOTAL) - p_k_f

    key0 = key_ref[...]
    s16n_ref[...] = jnp.where(
        key0 > 0, jax.lax.shift_right_arithmetic(key0, 16), -1
    ).astype(jnp.int16)
    s16p_ref[...] = jnp.where(
        key0 < 0, jax.lax.shift_right_arithmetic(-key0, 16), -1
    ).astype(jnp.int16)

    def count16(ref, cand):
        ind = jnp.where(ref[...] >= cand.astype(jnp.int16),
                        jnp.bfloat16(1), jnp.bfloat16(0))
        psum = jnp.sum(ind, axis=0, dtype=jnp.bfloat16)
        return jnp.sum(psum.astype(jnp.float32))

    def body_hi(i, carry):
        sel_hn, sel_hp = carry
        m = jax.lax.shift_left(jnp.int32(1), 14 - i)
        c_n = jax.lax.bitwise_or(sel_hn, m)
        c_p = jax.lax.bitwise_or(sel_hp, m)
        sel_hn = jnp.where(count16(s16n_ref, c_n) >= n_k_f, c_n, sel_hn)
        sel_hp = jnp.where(count16(s16p_ref, c_p) >= p_k_f, c_p, sel_hp)
        return sel_hn, sel_hp

    sel_hn, sel_hp = jax.lax.fori_loop(
        0, 15, body_hi, (jnp.int32(0), jnp.int32(0)))

    key1 = key_ref[...]
    posm = key1 > 0
    bits_abs = jnp.abs(key1)
    hi = jax.lax.shift_right_logical(bits_abs, 16)
    lo_off = jnp.bitwise_and(bits_abs, 65535) - 32768
    cnt_ab_n = jnp.sum(jnp.where(posm & (hi > sel_hn), 1.0, 0.0))
    cnt_ab_p = jnp.sum(jnp.where((~posm) & (hi > sel_hp), 1.0, 0.0))
    s16n_ref[...] = jnp.where(posm & (hi == sel_hn),
                              lo_off, -32768).astype(jnp.int16)
    s16p_ref[...] = jnp.where((~posm) & (hi == sel_hp),
                              lo_off, -32768).astype(jnp.int16)
    k2n_f = n_k_f - cnt_ab_n
    k2p_f = p_k_f - cnt_ab_p

    def body_lo(i, carry):
        sel_ln, sel_lp = carry
        m = jax.lax.shift_left(jnp.int32(1), 15 - i)
        c_n = jax.lax.bitwise_or(sel_ln, m)
        c_p = jax.lax.bitwise_or(sel_lp, m)
        sel_ln = jnp.where(count16(s16n_ref, c_n - 32768) >= k2n_f,
                           c_n, sel_ln)
        sel_lp = jnp.where(count16(s16p_ref, c_p - 32768) >= k2p_f,
                           c_p, sel_lp)
        return sel_ln, sel_lp

    sel_ln, sel_lp = jax.lax.fori_loop(
        0, 16, body_lo, (jnp.int32(0), jnp.int32(0)))

    sel_n = jax.lax.bitwise_or(jax.lax.shift_left(sel_hn, 16), sel_ln)
    sel_p = jax.lax.bitwise_or(jax.lax.shift_left(sel_hp, 16), sel_lp)
    thr_n = jax.lax.bitcast_convert_type(sel_n, jnp.float32)
    thr_p = jax.lax.bitcast_convert_type(sel_p, jnp.float32)

    loss2 = loss_ref[...]
    corr2 = corr_ref[...]
    neg = label_ref[...] == 0
    u0 = jnp.where(neg, loss2, 0.0)
    u1 = jnp.where(neg, 0.0, loss2)
    keep = (u0 < thr_n) & (u1 < thr_p)
    final = jnp.where(keep, loss2, corr2)
    out_final_ref[0, 0] = jnp.sum(final)
    out_loss_ref[0, 0] = jnp.sum(loss2)


def kernel(cls_score, label, class_freq, neg_class_freq, epoch=1):
    train_num = (class_freq[0] + neg_class_freq[0]).reshape(1, 1)
    cf = class_freq.reshape(1, C)
    sums = pl.pallas_call(
        _main_kernel,
        out_shape=[
            jax.ShapeDtypeStruct((1, 1), jnp.float32),
            jax.ShapeDtypeStruct((1, 1), jnp.float32),
        ],
        in_specs=[
            pl.BlockSpec(memory_space=pltpu.SMEM),
            pl.BlockSpec(memory_space=pltpu.VMEM),
            pl.BlockSpec(memory_space=pltpu.VMEM),
            pl.BlockSpec(memory_space=pltpu.VMEM),
        ],
        out_specs=[
            pl.BlockSpec(memory_space=pltpu.SMEM),
            pl.BlockSpec(memory_space=pltpu.SMEM),
        ],
        scratch_shapes=[
            pltpu.VMEM((B, C), jnp.float32),
            pltpu.VMEM((B, C), jnp.float32),
            pltpu.VMEM((B, C), jnp.int32),
            pltpu.VMEM((B, C), jnp.int16),
            pltpu.VMEM((B, C), jnp.int16),
        ],
        compiler_params=pltpu.CompilerParams(
            vmem_limit_bytes=100 * 1024 * 1024,
        ),
    )(train_num, cls_score, label, cf)
    inv_n = 1.0 / float(B * C)
    mean_final = sums[0][0, 0] * inv_n
    mean_loss = sums[1][0, 0] * inv_n
    return jnp.where(epoch == 0, mean_loss, mean_final)


# speedup vs baseline: 1.1436x; 1.0359x over previous
import math

import jax
import jax.numpy as jnp
from jax.experimental import pallas as pl
from jax.experimental.pallas import tpu as pltpu

B, C = 128, 8192
NEG_SCALE = 5.0
INIT_BIAS = 0.1
MAP_ALPHA, MAP_BETA, MAP_GAMMA = 10.0, 0.2, 0.1
FOCAL_GAMMA = 2.0
BALANCE_PARAM = 2.0
LOSS_WEIGHT = 1.0

CLEAN_RATE = 0.9
K_TOTAL = math.ceil(B * C * (1.0 - CLEAN_RATE))
P_K_MAX = math.ceil(K_TOTAL * 0.1)


def _main_kernel(tn_ref, score_ref, label_ref, cf_ref,
                 out_final_ref, out_loss_ref,
                 loss_ref, corr_ref, key_ref, s16n_ref, s16p_ref):
    score = score_ref[...]
    lab_i = label_ref[...]
    cf = cf_ref[...]
    tn = tn_ref[0, 0]

    init_bias = -jnp.log(tn / cf - 1.0) * (INIT_BIAS / NEG_SCALE)
    freq_inv = 1.0 / cf
    labf = jnp.maximum(lab_i, 0).astype(jnp.float32)

    def loss_an(sb, lab):
        rr = jnp.sum(lab * freq_inv, axis=1, keepdims=True)
        pw = freq_inv * (1.0 / rr)
        w = jax.nn.sigmoid(MAP_BETA * (pw - MAP_GAMMA)) + MAP_ALPHA
        logits = sb * (NEG_SCALE - (NEG_SCALE - 1.0) * lab)
        w = w * ((1.0 / NEG_SCALE)
                 + (1.0 - 1.0 / NEG_SCALE) * lab)
        bce = (jnp.maximum(logits, 0.0) - logits * lab
               + jnp.log1p(jnp.exp(-jnp.abs(logits))))
        pt = jnp.exp(-bce)
        om = 1.0 - pt
        return (LOSS_WEIGHT * BALANCE_PARAM) * (om * om * (w * bce))

    s1 = score + init_bias
    loss = loss_an(s1, labf)
    corr = loss_an(s1 + init_bias, 1.0 - labf)
    loss_ref[...] = loss
    corr_ref[...] = corr
    bits = jax.lax.bitcast_convert_type(loss, jnp.int32)
    neg0 = lab_i == 0
    key_ref[...] = jnp.where(neg0, bits, -bits)

    pos_f = jnp.sum(labf)
    p_k_f = jnp.minimum(jnp.float32(P_K_MAX), pos_f)
    n_k_f = jnp.float32(K_TOTAL) - p_k_f

    key0 = key_ref[...]
    s16n_ref[...] = jnp.where(
        key0 > 0, jax.lax.shift_right_arithmetic(key0, 16), -1
    ).astype(jnp.int16)
    s16p_ref[...] = jnp.where(
        key0 < 0, jax.lax.shift_right_arithmetic(-key0, 16), -1
    ).astype(jnp.int16)

    def count16(ref, cand):
        ind = jnp.where(ref[...] >= cand.astype(jnp.int16),
                        jnp.bfloat16(1), jnp.bfloat16(0))
        psum = jnp.sum(ind, axis=0, dtype=jnp.bfloat16)
        return jnp.sum(psum.astype(jnp.float32))

    def body_hi(i, carry):
        sel_hn, sel_hp = carry
        m = jax.lax.shift_left(jnp.int32(1), 14 - i)
        c_n = jax.lax.bitwise_or(sel_hn, m)
        c_p = jax.lax.bitwise_or(sel_hp, m)
        sel_hn = jnp.where(count16(s16n_ref, c_n) >= n_k_f, c_n, sel_hn)
        sel_hp = jnp.where(count16(s16p_ref, c_p) >= p_k_f, c_p, sel_hp)
        return sel_hn, sel_hp

    sel_hn, sel_hp = jax.lax.fori_loop(
        0, 15, body_hi, (jnp.int32(0), jnp.int32(0)))

    key1 = key_ref[...]
    posm = key1 > 0
    bits_abs = jnp.abs(key1)
    hi = jax.lax.shift_right_logical(bits_abs, 16)
    lo_off = jnp.bitwise_and(bits_abs, 65535) - 32768
    cnt_ab_n = jnp.sum(jnp.where(posm & (hi > sel_hn), 1.0, 0.0))
    cnt_ab_p = jnp.sum(jnp.where((~posm) & (hi > sel_hp), 1.0, 0.0))
    s16n_ref[...] = jnp.where(posm & (hi == sel_hn),
                              lo_off, -32768).astype(jnp.int16)
    s16p_ref[...] = jnp.where((~posm) & (hi == sel_hp),
                              lo_off, -32768).astype(jnp.int16)
    k2n_f = n_k_f - cnt_ab_n
    k2p_f = p_k_f - cnt_ab_p

    def body_lo(i, carry):
        sel_ln, sel_lp = carry
        m = jax.lax.shift_left(jnp.int32(1), 15 - i)
        c_n = jax.lax.bitwise_or(sel_ln, m)
        c_p = jax.lax.bitwise_or(sel_lp, m)
        sel_ln = jnp.where(count16(s16n_ref, c_n - 32768) >= k2n_f,
                           c_n, sel_ln)
        sel_lp = jnp.where(count16(s16p_ref, c_p - 32768) >= k2p_f,
                           c_p, sel_lp)
        return sel_ln, sel_lp

    sel_ln, sel_lp = jax.lax.fori_loop(
        0, 16, body_lo, (jnp.int32(0), jnp.int32(0)))

    sel_n = jax.lax.bitwise_or(jax.lax.shift_left(sel_hn, 16), sel_ln)
    sel_p = jax.lax.bitwise_or(jax.lax.shift_left(sel_hp, 16), sel_lp)
    thr_n = jax.lax.bitcast_convert_type(sel_n, jnp.float32)
    thr_p = jax.lax.bitcast_convert_type(sel_p, jnp.float32)

    loss2 = loss_ref[...]
    corr2 = corr_ref[...]
    neg = label_ref[...] == 0
    u0 = jnp.where(neg, loss2, 0.0)
    u1 = jnp.where(neg, 0.0, loss2)
    keep = (u0 < thr_n) & (u1 < thr_p)
    final = jnp.where(keep, loss2, corr2)
    out_final_ref[0, 0] = jnp.sum(final)
    out_loss_ref[0, 0] = jnp.sum(loss2)


def kernel(cls_score, label, class_freq, neg_class_freq, epoch=1):
    train_num = (class_freq[0] + neg_class_freq[0]).reshape(1, 1)
    cf = class_freq.reshape(1, C)
    sums = pl.pallas_call(
        _main_kernel,
        out_shape=[
            jax.ShapeDtypeStruct((1, 1), jnp.float32),
            jax.ShapeDtypeStruct((1, 1), jnp.float32),
        ],
        in_specs=[
            pl.BlockSpec(memory_space=pltpu.SMEM),
            pl.BlockSpec(memory_space=pltpu.VMEM),
            pl.BlockSpec(memory_space=pltpu.VMEM),
            pl.BlockSpec(memory_space=pltpu.VMEM),
        ],
        out_specs=[
            pl.BlockSpec(memory_space=pltpu.SMEM),
            pl.BlockSpec(memory_space=pltpu.SMEM),
        ],
        scratch_shapes=[
            pltpu.VMEM((B, C), jnp.float32),
            pltpu.VMEM((B, C), jnp.float32),
            pltpu.VMEM((B, C), jnp.int32),
            pltpu.VMEM((B, C), jnp.int16),
            pltpu.VMEM((B, C), jnp.int16),
        ],
        compiler_params=pltpu.CompilerParams(
            vmem_limit_bytes=100 * 1024 * 1024,
        ),
    )(train_num, cls_score, label, cf)
    inv_n = 1.0 / float(B * C)
    mean_final = sums[0][0, 0] * inv_n
    mean_loss = sums[1][0, 0] * inv_n
    return jnp.where(epoch == 0, mean_loss, mean_final)


# speedup vs baseline: 1.1454x; 1.0015x over previous
import math

import jax
import jax.numpy as jnp
from jax.experimental import pallas as pl
from jax.experimental.pallas import tpu as pltpu

B, C = 128, 8192
NEG_SCALE = 5.0
INIT_BIAS = 0.1
MAP_ALPHA, MAP_BETA, MAP_GAMMA = 10.0, 0.2, 0.1
FOCAL_GAMMA = 2.0
BALANCE_PARAM = 2.0
LOSS_WEIGHT = 1.0

CLEAN_RATE = 0.9
K_TOTAL = math.ceil(B * C * (1.0 - CLEAN_RATE))
P_K_MAX = math.ceil(K_TOTAL * 0.1)


def _main_kernel(tn_ref, score_ref, label_ref, cf_ref,
                 out_final_ref, out_loss_ref,
                 loss_ref, corr_ref, key_ref, s16n_ref, s16p_ref):
    score = score_ref[...]
    lab_i = label_ref[...]
    cf = cf_ref[...]
    tn = tn_ref[0, 0]

    init_bias = -jnp.log(tn / cf - 1.0) * (INIT_BIAS / NEG_SCALE)
    freq_inv = 1.0 / cf
    labf = jnp.maximum(lab_i, 0).astype(jnp.float32)

    def loss_an(sb, lab, rr):
        pw = freq_inv * (1.0 / rr)
        w = jax.nn.sigmoid(MAP_BETA * (pw - MAP_GAMMA)) + MAP_ALPHA
        logits = sb * (NEG_SCALE - (NEG_SCALE - 1.0) * lab)
        w = w * ((1.0 / NEG_SCALE)
                 + (1.0 - 1.0 / NEG_SCALE) * lab)
        bce = (jnp.maximum(logits, 0.0) - logits * lab
               + jnp.log1p(jnp.exp(-jnp.abs(logits))))
        pt = jnp.exp(-bce)
        om = 1.0 - pt
        return (LOSS_WEIGHT * BALANCE_PARAM) * (om * om * (w * bce))

    s1 = score + init_bias
    rr1 = jnp.sum(labf * freq_inv, axis=1, keepdims=True)
    loss = loss_an(s1, labf, rr1)
    corr = loss_an(s1 + init_bias, 1.0 - labf,
                   jnp.sum(freq_inv, axis=1, keepdims=True) - rr1)
    loss_ref[...] = loss
    corr_ref[...] = corr
    bits = jax.lax.bitcast_convert_type(loss, jnp.int32)
    neg0 = lab_i == 0
    key_ref[...] = jnp.where(neg0, bits, -bits)

    pos_f = jnp.sum(labf)
    p_k_f = jnp.minimum(jnp.float32(P_K_MAX), pos_f)
    n_k_f = jnp.float32(K_TOTAL) - p_k_f

    key0 = key_ref[...]
    s16n_ref[...] = jnp.where(
        key0 > 0, jax.lax.shift_right_arithmetic(key0, 16), -1
    ).astype(jnp.int16)
    s16p_ref[...] = jnp.where(
        key0 < 0, jax.lax.shift_right_arithmetic(-key0, 16), -1
    ).astype(jnp.int16)

    def count16(ref, cand):
        ind = jnp.where(ref[...] >= cand.astype(jnp.int16),
                        jnp.bfloat16(1), jnp.bfloat16(0))
        psum = jnp.sum(ind, axis=0, dtype=jnp.bfloat16)
        return jnp.sum(psum.astype(jnp.float32))

    def body_hi(i, carry):
        sel_hn, sel_hp = carry
        m = jax.lax.shift_left(jnp.int32(1), 14 - i)
        c_n = jax.lax.bitwise_or(sel_hn, m)
        c_p = jax.lax.bitwise_or(sel_hp, m)
        sel_hn = jnp.where(count16(s16n_ref, c_n) >= n_k_f, c_n, sel_hn)
        sel_hp = jnp.where(count16(s16p_ref, c_p) >= p_k_f, c_p, sel_hp)
        return sel_hn, sel_hp

    sel_hn, sel_hp = jax.lax.fori_loop(
        0, 15, body_hi, (jnp.int32(0), jnp.int32(0)))

    key1 = key_ref[...]
    posm = key1 > 0
    bits_abs = jnp.abs(key1)
    hi = jax.lax.shift_right_logical(bits_abs, 16)
    lo_off = jnp.bitwise_and(bits_abs, 65535) - 32768
    cnt_ab_n = jnp.sum(jnp.where(posm & (hi > sel_hn), 1.0, 0.0))
    cnt_ab_p = jnp.sum(jnp.where((~posm) & (hi > sel_hp), 1.0, 0.0))
    s16n_ref[...] = jnp.where(posm & (hi == sel_hn),
                              lo_off, -32768).astype(jnp.int16)
    s16p_ref[...] = jnp.where((~posm) & (hi == sel_hp),
                              lo_off, -32768).astype(jnp.int16)
    k2n_f = n_k_f - cnt_ab_n
    k2p_f = p_k_f - cnt_ab_p

    def body_lo(i, carry):
        sel_ln, sel_lp = carry
        m = jax.lax.shift_left(jnp.int32(1), 15 - i)
        c_n = jax.lax.bitwise_or(sel_ln, m)
        c_p = jax.lax.bitwise_or(sel_lp, m)
        sel_ln = jnp.where(count16(s16n_ref, c_n - 32768) >= k2n_f,
                           c_n, sel_ln)
        sel_lp = jnp.where(count16(s16p_ref, c_p - 32768) >= k2p_f,
                           c_p, sel_lp)
        return sel_ln, sel_lp

    sel_ln, sel_lp = jax.lax.fori_loop(
        0, 16, body_lo, (jnp.int32(0), jnp.int32(0)))

    sel_n = jax.lax.bitwise_or(jax.lax.shift_left(sel_hn, 16), sel_ln)
    sel_p = jax.lax.bitwise_or(jax.lax.shift_left(sel_hp, 16), sel_lp)
    thr_n = jax.lax.bitcast_convert_type(sel_n, jnp.float32)
    thr_p = jax.lax.bitcast_convert_type(sel_p, jnp.float32)

    loss2 = loss_ref[...]
    corr2 = corr_ref[...]
    neg = label_ref[...] == 0
    u0 = jnp.where(neg, loss2, 0.0)
    u1 = jnp.where(neg, 0.0, loss2)
    keep = (u0 < thr_n) & (u1 < thr_p)
    final = jnp.where(keep, loss2, corr2)
    out_final_ref[0, 0] = jnp.sum(final)
    out_loss_ref[0, 0] = jnp.sum(loss2)


def kernel(cls_score, label, class_freq, neg_class_freq, epoch=1):
    train_num = (class_freq[0] + neg_class_freq[0]).reshape(1, 1)
    cf = class_freq.reshape(1, C)
    sums = pl.pallas_call(
        _main_kernel,
        out_shape=[
            jax.ShapeDtypeStruct((1, 1), jnp.float32),
            jax.ShapeDtypeStruct((1, 1), jnp.float32),
        ],
        in_specs=[
            pl.BlockSpec(memory_space=pltpu.SMEM),
            pl.BlockSpec(memory_space=pltpu.VMEM),
            pl.BlockSpec(memory_space=pltpu.VMEM),
            pl.BlockSpec(memory_space=pltpu.VMEM),
        ],
        out_specs=[
            pl.BlockSpec(memory_space=pltpu.SMEM),
            pl.BlockSpec(memory_space=pltpu.SMEM),
        ],
        scratch_shapes=[
            pltpu.VMEM((B, C), jnp.float32),
            pltpu.VMEM((B, C), jnp.float32),
            pltpu.VMEM((B, C), jnp.int32),
            pltpu.VMEM((B, C), jnp.int16),
            pltpu.VMEM((B, C), jnp.int16),
        ],
        compiler_params=pltpu.CompilerParams(
            vmem_limit_bytes=100 * 1024 * 1024,
        ),
    )(train_num, cls_score, label, cf)
    inv_n = 1.0 / float(B * C)
    mean_final = sums[0][0, 0] * inv_n
    mean_loss = sums[1][0, 0] * inv_n
    return jnp.where(epoch == 0, mean_loss, mean_final)


# speedup vs baseline: 1.1534x; 1.0071x over previous
import math

import jax
import jax.numpy as jnp
from jax.experimental import pallas as pl
from jax.experimental.pallas import tpu as pltpu

B, C = 128, 8192
NEG_SCALE = 5.0
INIT_BIAS = 0.1
MAP_ALPHA, MAP_BETA, MAP_GAMMA = 10.0, 0.2, 0.1
FOCAL_GAMMA = 2.0
BALANCE_PARAM = 2.0
LOSS_WEIGHT = 1.0

CLEAN_RATE = 0.9
K_TOTAL = math.ceil(B * C * (1.0 - CLEAN_RATE))
P_K_MAX = math.ceil(K_TOTAL * 0.1)


def _main_kernel(tn_ref, score_ref, label_ref, cf_ref,
                 out_final_ref, out_loss_ref,
                 loss_ref, corr_ref, key_ref, s16n_ref, s16p_ref):
    score = score_ref[...]
    lab_i = label_ref[...]
    cf = cf_ref[...]
    tn = tn_ref[0, 0]

    init_bias = -jnp.log(tn / cf - 1.0) * (INIT_BIAS / NEG_SCALE)
    freq_inv = 1.0 / cf
    labf = jnp.maximum(lab_i, 0).astype(jnp.float32)

    def loss_an(sb, lab, rr):
        pw = freq_inv * (1.0 / rr)
        w = jax.nn.sigmoid(MAP_BETA * (pw - MAP_GAMMA)) + MAP_ALPHA
        logits = sb * (NEG_SCALE - (NEG_SCALE - 1.0) * lab)
        w = w * ((1.0 / NEG_SCALE)
                 + (1.0 - 1.0 / NEG_SCALE) * lab)
        bce = (jnp.maximum(logits, 0.0) - logits * lab
               + jnp.log1p(jnp.exp(-jnp.abs(logits))))
        pt = jnp.exp(-bce)
        om = 1.0 - pt
        return (LOSS_WEIGHT * BALANCE_PARAM) * (om * om * (w * bce))

    s1 = score + init_bias
    rr1 = jnp.sum(labf * freq_inv, axis=1, keepdims=True)
    loss = loss_an(s1, labf, rr1)
    corr = loss_an(s1 + init_bias, 1.0 - labf,
                   jnp.sum(freq_inv, axis=1, keepdims=True) - rr1)
    loss_ref[...] = loss
    corr_ref[...] = corr
    bits = jax.lax.bitcast_convert_type(loss, jnp.int32)
    neg0 = lab_i == 0
    key_ref[...] = jnp.where(neg0, bits, -bits)

    pos_f = jnp.sum(labf)
    p_k_f = jnp.minimum(jnp.float32(P_K_MAX), pos_f)
    n_k_f = jnp.float32(K_TOTAL) - p_k_f

    key0 = key_ref[...]
    hi_n = jax.lax.shift_right_arithmetic(key0, 16)
    hi_p = jax.lax.shift_right_arithmetic(-key0, 16)
    s16n_ref[...] = jnp.where(
        key0 > 0, hi_n, jax.lax.bitwise_not(hi_p)).astype(jnp.int16)

    def count16(ref, cand):
        ind = jnp.where(ref[...] >= cand.astype(jnp.int16),
                        jnp.bfloat16(1), jnp.bfloat16(0))
        psum = jnp.sum(ind, axis=0, dtype=jnp.bfloat16)
        return jnp.sum(psum.astype(jnp.float32))

    def body_hi(i, carry):
        sel_hn, sel_hp = carry
        m = jax.lax.shift_left(jnp.int32(1), 14 - i)
        c_n = jax.lax.bitwise_or(sel_hn, m)
        c_p = jax.lax.bitwise_or(sel_hp, m)
        s = s16n_ref[...]
        ind_n = jnp.where(s >= c_n.astype(jnp.int16),
                          jnp.bfloat16(1), jnp.bfloat16(0))
        ind_p = jnp.where(s < (-c_p).astype(jnp.int16),
                          jnp.bfloat16(1), jnp.bfloat16(0))
        cnt_n = jnp.sum(jnp.sum(ind_n, axis=0, dtype=jnp.bfloat16)
                        .astype(jnp.float32))
        cnt_p = jnp.sum(jnp.sum(ind_p, axis=0, dtype=jnp.bfloat16)
                        .astype(jnp.float32))
        sel_hn = jnp.where(cnt_n >= n_k_f, c_n, sel_hn)
        sel_hp = jnp.where(cnt_p >= p_k_f, c_p, sel_hp)
        return sel_hn, sel_hp

    sel_hn, sel_hp = jax.lax.fori_loop(
        0, 15, body_hi, (jnp.int32(0), jnp.int32(0)))

    key1 = key_ref[...]
    posm = key1 > 0
    bits_abs = jnp.abs(key1)
    hi = jax.lax.shift_right_logical(bits_abs, 16)
    lo_off = jnp.bitwise_and(bits_abs, 65535) - 32768
    cnt_ab_n = jnp.sum(jnp.where(posm & (hi > sel_hn), 1.0, 0.0))
    cnt_ab_p = jnp.sum(jnp.where((~posm) & (hi > sel_hp), 1.0, 0.0))
    s16n_ref[...] = jnp.where(posm & (hi == sel_hn),
                              lo_off, -32768).astype(jnp.int16)
    s16p_ref[...] = jnp.where((~posm) & (hi == sel_hp),
                              lo_off, -32768).astype(jnp.int16)
    k2n_f = n_k_f - cnt_ab_n
    k2p_f = p_k_f - cnt_ab_p

    def body_lo(i, carry):
        sel_ln, sel_lp = carry
        m = jax.lax.shift_left(jnp.int32(1), 15 - i)
        c_n = jax.lax.bitwise_or(sel_ln, m)
        c_p = jax.lax.bitwise_or(sel_lp, m)
        sel_ln = jnp.where(count16(s16n_ref, c_n - 32768) >= k2n_f,
                           c_n, sel_ln)
        sel_lp = jnp.where(count16(s16p_ref, c_p - 32768) >= k2p_f,
                           c_p, sel_lp)
        return sel_ln, sel_lp

    sel_ln, sel_lp = jax.lax.fori_loop(
        0, 16, body_lo, (jnp.int32(0), jnp.int32(0)))

    sel_n = jax.lax.bitwise_or(jax.lax.shift_left(sel_hn, 16), sel_ln)
    sel_p = jax.lax.bitwise_or(jax.lax.shift_left(sel_hp, 16), sel_lp)
    thr_n = jax.lax.bitcast_convert_type(sel_n, jnp.float32)
    thr_p = jax.lax.bitcast_convert_type(sel_p, jnp.float32)

    loss2 = loss_ref[...]
    corr2 = corr_ref[...]
    neg = label_ref[...] == 0
    u0 = jnp.where(neg, loss2, 0.0)
    u1 = jnp.where(neg, 0.0, loss2)
    keep = (u0 < thr_n) & (u1 < thr_p)
    final = jnp.where(keep, loss2, corr2)
    out_final_ref[0, 0] = jnp.sum(final)
    out_loss_ref[0, 0] = jnp.sum(loss2)


def kernel(cls_score, label, class_freq, neg_class_freq, epoch=1):
    train_num = (class_freq[0] + neg_class_freq[0]).reshape(1, 1)
    cf = class_freq.reshape(1, C)
    sums = pl.pallas_call(
        _main_kernel,
        out_shape=[
            jax.ShapeDtypeStruct((1, 1), jnp.float32),
            jax.ShapeDtypeStruct((1, 1), jnp.float32),
        ],
        in_specs=[
            pl.BlockSpec(memory_space=pltpu.SMEM),
            pl.BlockSpec(memory_space=pltpu.VMEM),
            pl.BlockSpec(memory_space=pltpu.VMEM),
            pl.BlockSpec(memory_space=pltpu.VMEM),
        ],
        out_specs=[
            pl.BlockSpec(memory_space=pltpu.SMEM),
            pl.BlockSpec(memory_space=pltpu.SMEM),
        ],
        scratch_shapes=[
            pltpu.VMEM((B, C), jnp.float32),
            pltpu.VMEM((B, C), jnp.float32),
            pltpu.VMEM((B, C), jnp.int32),
            pltpu.VMEM((B, C), jnp.int16),
            pltpu.VMEM((B, C), jnp.int16),
        ],
        compiler_params=pltpu.CompilerParams(
            vmem_limit_bytes=100 * 1024 * 1024,
        ),
    )(train_num, cls_score, label, cf)
    inv_n = 1.0 / float(B * C)
    mean_final = sums[0][0, 0] * inv_n
    mean_loss = sums[1][0, 0] * inv_n
    return jnp.where(epoch == 0, mean_loss, mean_final)


# speedup vs baseline: 1.1905x; 1.0322x over previous
import math

import jax
import jax.numpy as jnp
from jax.experimental import pallas as pl
from jax.experimental.pallas import tpu as pltpu

B, C = 128, 8192
NEG_SCALE = 5.0
INIT_BIAS = 0.1
MAP_ALPHA, MAP_BETA, MAP_GAMMA = 10.0, 0.2, 0.1
FOCAL_GAMMA = 2.0
BALANCE_PARAM = 2.0
LOSS_WEIGHT = 1.0

CLEAN_RATE = 0.9
K_TOTAL = math.ceil(B * C * (1.0 - CLEAN_RATE))
P_K_MAX = math.ceil(K_TOTAL * 0.1)


def _main_kernel(tn_ref, score_ref, label_ref, cf_ref,
                 out_final_ref, out_loss_ref,
                 loss_ref, corr_ref, key_ref, s16n_ref, s16p_ref):
    score = score_ref[...]
    lab_i = label_ref[...]
    cf = cf_ref[...]
    tn = tn_ref[0, 0]

    init_bias = -jnp.log(tn / cf - 1.0) * (INIT_BIAS / NEG_SCALE)
    freq_inv = 1.0 / cf
    labf = jnp.maximum(lab_i, 0).astype(jnp.float32)

    def loss_an(sb, lab, rr):
        pw = freq_inv * (1.0 / rr)
        w = jax.nn.sigmoid(MAP_BETA * (pw - MAP_GAMMA)) + MAP_ALPHA
        logits = sb * (NEG_SCALE - (NEG_SCALE - 1.0) * lab)
        w = w * ((1.0 / NEG_SCALE)
                 + (1.0 - 1.0 / NEG_SCALE) * lab)
        bce = (jnp.maximum(logits, 0.0) - logits * lab
               + jnp.log(1.0 + jnp.exp(-jnp.abs(logits))))
        pt = jnp.exp(-bce)
        om = 1.0 - pt
        return (LOSS_WEIGHT * BALANCE_PARAM) * (om * om * (w * bce))

    s1 = score + init_bias
    rr1 = jnp.sum(labf * freq_inv, axis=1, keepdims=True)
    loss = loss_an(s1, labf, rr1)
    corr = loss_an(s1 + init_bias, 1.0 - labf,
                   jnp.sum(freq_inv, axis=1, keepdims=True) - rr1)
    loss_ref[...] = loss
    corr_ref[...] = corr
    bits = jax.lax.bitcast_convert_type(loss, jnp.int32)
    neg0 = lab_i == 0
    key_ref[...] = jnp.where(neg0, bits, -bits)

    pos_f = jnp.sum(labf)
    p_k_f = jnp.minimum(jnp.float32(P_K_MAX), pos_f)
    n_k_f = jnp.float32(K_TOTAL) - p_k_f

    key0 = key_ref[...]
    hi_n = jax.lax.shift_right_arithmetic(key0, 16)
    hi_p = jax.lax.shift_right_arithmetic(-key0, 16)
    s16n_ref[...] = jnp.where(
        key0 > 0, hi_n, jax.lax.bitwise_not(hi_p)).astype(jnp.int16)

    def count16(ref, cand):
        ind = jnp.where(ref[...] >= cand.astype(jnp.int16),
                        jnp.bfloat16(1), jnp.bfloat16(0))
        psum = jnp.sum(ind, axis=0, dtype=jnp.bfloat16)
        return jnp.sum(psum.astype(jnp.float32))

    def body_hi(i, carry):
        sel_hn, sel_hp = carry
        m = jax.lax.shift_left(jnp.int32(1), 14 - i)
        c_n = jax.lax.bitwise_or(sel_hn, m)
        c_p = jax.lax.bitwise_or(sel_hp, m)
        s = s16n_ref[...]
        ind_n = jnp.where(s >= c_n.astype(jnp.int16),
                          jnp.bfloat16(1), jnp.bfloat16(0))
        ind_p = jnp.where(s < (-c_p).astype(jnp.int16),
                          jnp.bfloat16(1), jnp.bfloat16(0))
        cnt_n = jnp.sum(jnp.sum(ind_n, axis=0, dtype=jnp.bfloat16)
                        .astype(jnp.float32))
        cnt_p = jnp.sum(jnp.sum(ind_p, axis=0, dtype=jnp.bfloat16)
                        .astype(jnp.float32))
        sel_hn = jnp.where(cnt_n >= n_k_f, c_n, sel_hn)
        sel_hp = jnp.where(cnt_p >= p_k_f, c_p, sel_hp)
        return sel_hn, sel_hp

    sel_hn, sel_hp = jax.lax.fori_loop(
        0, 15, body_hi, (jnp.int32(0), jnp.int32(0)))

    key1 = key_ref[...]
    posm = key1 > 0
    bits_abs = jnp.abs(key1)
    hi = jax.lax.shift_right_logical(bits_abs, 16)
    lo_off = jnp.bitwise_and(bits_abs, 65535) - 32768
    cnt_ab_n = jnp.sum(jnp.where(posm & (hi > sel_hn), 1.0, 0.0))
    cnt_ab_p = jnp.sum(jnp.where((~posm) & (hi > sel_hp), 1.0, 0.0))
    s16n_ref[...] = jnp.where(posm & (hi == sel_hn),
                              lo_off, -32768).astype(jnp.int16)
    s16p_ref[...] = jnp.where((~posm) & (hi == sel_hp),
                              lo_off, -32768).astype(jnp.int16)
    k2n_f = n_k_f - cnt_ab_n
    k2p_f = p_k_f - cnt_ab_p

    def body_lo(i, carry):
        sel_ln, sel_lp = carry
        m = jax.lax.shift_left(jnp.int32(1), 15 - i)
        c_n = jax.lax.bitwise_or(sel_ln, m)
        c_p = jax.lax.bitwise_or(sel_lp, m)
        sel_ln = jnp.where(count16(s16n_ref, c_n - 32768) >= k2n_f,
                           c_n, sel_ln)
        sel_lp = jnp.where(count16(s16p_ref, c_p - 32768) >= k2p_f,
                           c_p, sel_lp)
        return sel_ln, sel_lp

    sel_ln, sel_lp = jax.lax.fori_loop(
        0, 16, body_lo, (jnp.int32(0), jnp.int32(0)))

    sel_n = jax.lax.bitwise_or(jax.lax.shift_left(sel_hn, 16), sel_ln)
    sel_p = jax.lax.bitwise_or(jax.lax.shift_left(sel_hp, 16), sel_lp)
    thr_n = jax.lax.bitcast_convert_type(sel_n, jnp.float32)
    thr_p = jax.lax.bitcast_convert_type(sel_p, jnp.float32)

    loss2 = loss_ref[...]
    corr2 = corr_ref[...]
    neg = label_ref[...] == 0
    u0 = jnp.where(neg, loss2, 0.0)
    u1 = jnp.where(neg, 0.0, loss2)
    keep = (u0 < thr_n) & (u1 < thr_p)
    final = jnp.where(keep, loss2, corr2)
    out_final_ref[0, 0] = jnp.sum(final)
    out_loss_ref[0, 0] = jnp.sum(loss2)


def kernel(cls_score, label, class_freq, neg_class_freq, epoch=1):
    train_num = (class_freq[0] + neg_class_freq[0]).reshape(1, 1)
    cf = class_freq.reshape(1, C)
    sums = pl.pallas_call(
        _main_kernel,
        out_shape=[
            jax.ShapeDtypeStruct((1, 1), jnp.float32),
            jax.ShapeDtypeStruct((1, 1), jnp.float32),
        ],
        in_specs=[
            pl.BlockSpec(memory_space=pltpu.SMEM),
            pl.BlockSpec(memory_space=pltpu.VMEM),
            pl.BlockSpec(memory_space=pltpu.VMEM),
            pl.BlockSpec(memory_space=pltpu.VMEM),
        ],
        out_specs=[
            pl.BlockSpec(memory_space=pltpu.SMEM),
            pl.BlockSpec(memory_space=pltpu.SMEM),
        ],
        scratch_shapes=[
            pltpu.VMEM((B, C), jnp.float32),
            pltpu.VMEM((B, C), jnp.float32),
            pltpu.VMEM((B, C), jnp.int32),
            pltpu.VMEM((B, C), jnp.int16),
            pltpu.VMEM((B, C), jnp.int16),
        ],
        compiler_params=pltpu.CompilerParams(
            vmem_limit_bytes=100 * 1024 * 1024,
        ),
    )(train_num, cls_score, label, cf)
    inv_n = 1.0 / float(B * C)
    mean_final = sums[0][0, 0] * inv_n
    mean_loss = sums[1][0, 0] * inv_n
    return jnp.where(epoch == 0, mean_loss, mean_final)


# speedup vs baseline: 1.1916x; 1.0009x over previous
import math

import jax
import jax.numpy as jnp
from jax.experimental import pallas as pl
from jax.experimental.pallas import tpu as pltpu

B, C = 128, 8192
NEG_SCALE = 5.0
INIT_BIAS = 0.1
MAP_ALPHA, MAP_BETA, MAP_GAMMA = 10.0, 0.2, 0.1
FOCAL_GAMMA = 2.0
BALANCE_PARAM = 2.0
LOSS_WEIGHT = 1.0

CLEAN_RATE = 0.9
K_TOTAL = math.ceil(B * C * (1.0 - CLEAN_RATE))
P_K_MAX = math.ceil(K_TOTAL * 0.1)


def _main_kernel(tn_ref, score_ref, label_ref, cf_ref,
                 out_final_ref, out_loss_ref,
                 loss_ref, corr_ref, key_ref, s16n_ref, s16p_ref):
    score = score_ref[...]
    lab_i = label_ref[...]
    cf = cf_ref[...]
    tn = tn_ref[0, 0]

    init_bias = -jnp.log(tn / cf - 1.0) * (INIT_BIAS / NEG_SCALE)
    freq_inv = 1.0 / cf
    labf = jnp.maximum(lab_i, 0).astype(jnp.float32)

    def loss_an(sb, lab, rr):
        pw = freq_inv * (1.0 / rr)
        w = 1.0 / (1.0 + jnp.exp(MAP_BETA * (MAP_GAMMA - pw))) + MAP_ALPHA
        logits = sb * (NEG_SCALE - (NEG_SCALE - 1.0) * lab)
        w = w * ((1.0 / NEG_SCALE)
                 + (1.0 - 1.0 / NEG_SCALE) * lab)
        bce = (jnp.maximum(logits, 0.0) - logits * lab
               + jnp.log(1.0 + jnp.exp(-jnp.abs(logits))))
        pt = jnp.exp(-bce)
        om = 1.0 - pt
        return (LOSS_WEIGHT * BALANCE_PARAM) * (om * om * (w * bce))

    s1 = score + init_bias
    rr1 = jnp.sum(labf * freq_inv, axis=1, keepdims=True)
    loss = loss_an(s1, labf, rr1)
    corr = loss_an(s1 + init_bias, 1.0 - labf,
                   jnp.sum(freq_inv, axis=1, keepdims=True) - rr1)
    loss_ref[...] = loss
    corr_ref[...] = corr
    bits = jax.lax.bitcast_convert_type(loss, jnp.int32)
    neg0 = lab_i == 0
    key_ref[...] = jnp.where(neg0, bits, -bits)

    pos_f = jnp.sum(labf)
    p_k_f = jnp.minimum(jnp.float32(P_K_MAX), pos_f)
    n_k_f = jnp.float32(K_TOTAL) - p_k_f

    key0 = key_ref[...]
    hi_n = jax.lax.shift_right_arithmetic(key0, 16)
    hi_p = jax.lax.shift_right_arithmetic(-key0, 16)
    s16n_ref[...] = jnp.where(
        key0 > 0, hi_n, jax.lax.bitwise_not(hi_p)).astype(jnp.int16)

    def count16(ref, cand):
        ind = jnp.where(ref[...] >= cand.astype(jnp.int16),
                        jnp.bfloat16(1), jnp.bfloat16(0))
        psum = jnp.sum(ind, axis=0, dtype=jnp.bfloat16)
        return jnp.sum(psum.astype(jnp.float32))

    def body_hi(i, carry):
        sel_hn, sel_hp = carry
        m = jax.lax.shift_left(jnp.int32(1), 14 - i)
        c_n = jax.lax.bitwise_or(sel_hn, m)
        c_p = jax.lax.bitwise_or(sel_hp, m)
        s = s16n_ref[...]
        ind_n = jnp.where(s >= c_n.astype(jnp.int16),
                          jnp.bfloat16(1), jnp.bfloat16(0))
        ind_p = jnp.where(s < (-c_p).astype(jnp.int16),
                          jnp.bfloat16(1), jnp.bfloat16(0))
        cnt_n = jnp.sum(jnp.sum(ind_n, axis=0, dtype=jnp.bfloat16)
                        .astype(jnp.float32))
        cnt_p = jnp.sum(jnp.sum(ind_p, axis=0, dtype=jnp.bfloat16)
                        .astype(jnp.float32))
        sel_hn = jnp.where(cnt_n >= n_k_f, c_n, sel_hn)
        sel_hp = jnp.where(cnt_p >= p_k_f, c_p, sel_hp)
        return sel_hn, sel_hp

    sel_hn, sel_hp = jax.lax.fori_loop(
        0, 15, body_hi, (jnp.int32(0), jnp.int32(0)))

    key1 = key_ref[...]
    posm = key1 > 0
    bits_abs = jnp.abs(key1)
    hi = jax.lax.shift_right_logical(bits_abs, 16)
    lo_off = jnp.bitwise_and(bits_abs, 65535) - 32768
    cnt_ab_n = jnp.sum(jnp.where(posm & (hi > sel_hn), 1.0, 0.0))
    cnt_ab_p = jnp.sum(jnp.where((~posm) & (hi > sel_hp), 1.0, 0.0))
    s16n_ref[...] = jnp.where(posm & (hi == sel_hn),
                              lo_off, -32768).astype(jnp.int16)
    s16p_ref[...] = jnp.where((~posm) & (hi == sel_hp),
                              lo_off, -32768).astype(jnp.int16)
    k2n_f = n_k_f - cnt_ab_n
    k2p_f = p_k_f - cnt_ab_p

    def body_lo(i, carry):
        sel_ln, sel_lp = carry
        m = jax.lax.shift_left(jnp.int32(1), 15 - i)
        c_n = jax.lax.bitwise_or(sel_ln, m)
        c_p = jax.lax.bitwise_or(sel_lp, m)
        sel_ln = jnp.where(count16(s16n_ref, c_n - 32768) >= k2n_f,
                           c_n, sel_ln)
        sel_lp = jnp.where(count16(s16p_ref, c_p - 32768) >= k2p_f,
                           c_p, sel_lp)
        return sel_ln, sel_lp

    sel_ln, sel_lp = jax.lax.fori_loop(
        0, 16, body_lo, (jnp.int32(0), jnp.int32(0)))

    sel_n = jax.lax.bitwise_or(jax.lax.shift_left(sel_hn, 16), sel_ln)
    sel_p = jax.lax.bitwise_or(jax.lax.shift_left(sel_hp, 16), sel_lp)
    thr_n = jax.lax.bitcast_convert_type(sel_n, jnp.float32)
    thr_p = jax.lax.bitcast_convert_type(sel_p, jnp.float32)

    loss2 = loss_ref[...]
    corr2 = corr_ref[...]
    neg = label_ref[...] == 0
    u0 = jnp.where(neg, loss2, 0.0)
    u1 = jnp.where(neg, 0.0, loss2)
    keep = (u0 < thr_n) & (u1 < thr_p)
    final = jnp.where(keep, loss2, corr2)
    out_final_ref[0, 0] = jnp.sum(final)
    out_loss_ref[0, 0] = jnp.sum(loss2)


def kernel(cls_score, label, class_freq, neg_class_freq, epoch=1):
    train_num = (class_freq[0] + neg_class_freq[0]).reshape(1, 1)
    cf = class_freq.reshape(1, C)
    sums = pl.pallas_call(
        _main_kernel,
        out_shape=[
            jax.ShapeDtypeStruct((1, 1), jnp.float32),
            jax.ShapeDtypeStruct((1, 1), jnp.float32),
        ],
        in_specs=[
            pl.BlockSpec(memory_space=pltpu.SMEM),
            pl.BlockSpec(memory_space=pltpu.VMEM),
            pl.BlockSpec(memory_space=pltpu.VMEM),
            pl.BlockSpec(memory_space=pltpu.VMEM),
        ],
        out_specs=[
            pl.BlockSpec(memory_space=pltpu.SMEM),
            pl.BlockSpec(memory_space=pltpu.SMEM),
        ],
        scratch_shapes=[
            pltpu.VMEM((B, C), jnp.float32),
            pltpu.VMEM((B, C), jnp.float32),
            pltpu.VMEM((B, C), jnp.int32),
            pltpu.VMEM((B, C), jnp.int16),
            pltpu.VMEM((B, C), jnp.int16),
        ],
        compiler_params=pltpu.CompilerParams(
            vmem_limit_bytes=100 * 1024 * 1024,
        ),
    )(train_num, cls_score, label, cf)
    inv_n = 1.0 / float(B * C)
    mean_final = sums[0][0, 0] * inv_n
    mean_loss = sums[1][0, 0] * inv_n
    return jnp.where(epoch == 0, mean_loss, mean_final)
